# baseline (speedup 1.0000x reference)
import functools

import jax
import jax.numpy as jnp
from jax.experimental import pallas as pl
from jax.experimental.pallas import tpu as pltpu

NQ = 4096
N = 100000
D = 128
NPAD = 100352
QB = 256
KB = 2048


def _dist_body(q_ref, qsq_ref, kT_ref, ksq_ref, out_ref):
    q = q_ref[...]
    kT = kT_ref[...]
    dot = jnp.dot(q, kT, preferred_element_type=jnp.float32)
    out_ref[...] = qsq_ref[...] + ksq_ref[...] - 2.0 * dot


def kernel(query, keys, W, k):
    q = query @ W.T
    qsq = jnp.sum(q * q, axis=1, keepdims=True)
    keysq = jnp.sum(keys * keys, axis=1)
    kT = jnp.concatenate(
        [keys.T, jnp.zeros((D, NPAD - N), jnp.float32)], axis=1)
    ksq_p = jnp.concatenate(
        [keysq, jnp.full((NPAD - N,), jnp.inf, jnp.float32)])
    ksq_p = ksq_p.reshape(1, NPAD)

    grid = (NPAD // KB, NQ // QB)
    dists = pl.pallas_call(
        _dist_body,
        grid=grid,
        in_specs=[
            pl.BlockSpec((QB, D), lambda kc, qb: (qb, 0)),
            pl.BlockSpec((QB, 1), lambda kc, qb: (qb, 0)),
            pl.BlockSpec((D, KB), lambda kc, qb: (0, kc)),
            pl.BlockSpec((1, KB), lambda kc, qb: (0, kc)),
        ],
        out_specs=pl.BlockSpec((QB, KB), lambda kc, qb: (qb, kc)),
        out_shape=jax.ShapeDtypeStruct((NQ, NPAD), jnp.float32),
    )(q, qsq, kT, ksq_p)

    neg_d, idx = jax.lax.top_k(-dists, 10)
    return (-neg_d, idx)


# speedup vs baseline: 1.6348x; 1.6348x over previous
import functools

import jax
import jax.numpy as jnp
from jax.experimental import pallas as pl
from jax.experimental.pallas import tpu as pltpu

NQ = 4096
N = 100000
D = 128
NPAD = 100352
QB = 256
KB = 2048
BIN = 64
NBINS = NPAD // BIN
BPC = KB // BIN
NSEL = 16


def _dist_body(q_ref, qsq_ref, kT_ref, ksq_ref, out_ref, bm_ref):
    q = q_ref[...]
    kT = kT_ref[...]
    dot = jnp.dot(q, kT, preferred_element_type=jnp.float32)
    d = qsq_ref[...] + ksq_ref[...] - 2.0 * dot
    out_ref[...] = d
    bm_ref[0] = jnp.min(d.reshape(QB, BPC, BIN), axis=2)


def _select_body(bm_ref, ids_ref):
    x = bm_ref[...]
    iota = jax.lax.broadcasted_iota(jnp.int32, (QB, NBINS), 1)
    ids_ref[...] = jnp.full((QB, NSEL), NBINS - 1, jnp.int32)
    for t in range(10):
        m = jnp.min(x, axis=1, keepdims=True)
        idx = jnp.min(jnp.where(x == m, iota, jnp.int32(2**30)),
                      axis=1, keepdims=True)
        ids_ref[:, t:t + 1] = idx
        x = jnp.where(iota == idx, jnp.inf, x)


def kernel(query, keys, W, k):
    q = query @ W.T
    qsq = jnp.sum(q * q, axis=1, keepdims=True)
    keysq = jnp.sum(keys * keys, axis=1)
    kT = jnp.concatenate(
        [keys.T, jnp.zeros((D, NPAD - N), jnp.float32)], axis=1)
    ksq_p = jnp.concatenate(
        [keysq, jnp.full((NPAD - N,), jnp.inf, jnp.float32)])
    ksq_p = ksq_p.reshape(1, NPAD)

    grid = (NPAD // KB, NQ // QB)
    dists, binmins = pl.pallas_call(
        _dist_body,
        grid=grid,
        in_specs=[
            pl.BlockSpec((QB, D), lambda kc, qb: (qb, 0)),
            pl.BlockSpec((QB, 1), lambda kc, qb: (qb, 0)),
            pl.BlockSpec((D, KB), lambda kc, qb: (0, kc)),
            pl.BlockSpec((1, KB), lambda kc, qb: (0, kc)),
        ],
        out_specs=[
            pl.BlockSpec((QB, KB), lambda kc, qb: (qb, kc)),
            pl.BlockSpec((1, QB, BPC), lambda kc, qb: (kc, qb, 0)),
        ],
        out_shape=[
            jax.ShapeDtypeStruct((NQ, NPAD), jnp.float32),
            jax.ShapeDtypeStruct((NPAD // KB, NQ, BPC), jnp.float32),
        ],
    )(q, qsq, kT, ksq_p)
    binmins = binmins.transpose(1, 0, 2).reshape(NQ, NBINS)

    ids = pl.pallas_call(
        _select_body,
        grid=(NQ // QB,),
        in_specs=[pl.BlockSpec((QB, NBINS), lambda qb: (qb, 0))],
        out_specs=pl.BlockSpec((QB, NSEL), lambda qb: (qb, 0)),
        out_shape=jax.ShapeDtypeStruct((NQ, NSEL), jnp.int32),
    )(binmins)

    flat = dists.reshape(NQ * NBINS, BIN)
    rowbase = jnp.arange(NQ, dtype=jnp.int32)[:, None] * NBINS
    vals = flat[(rowbase + ids).reshape(-1)].reshape(NQ, NSEL * BIN)
    cidx = (ids[:, :, None] * BIN
            + jnp.arange(BIN, dtype=jnp.int32)[None, None, :]
            ).reshape(NQ, NSEL * BIN)
    negv, pos = jax.lax.top_k(-vals, 10)
    out_idx = jnp.take_along_axis(cidx, pos, axis=1)
    return (-negv, out_idx)


# speedup vs baseline: 4.6459x; 2.8418x over previous
import functools

import jax
import jax.numpy as jnp
from jax import lax
from jax.experimental import pallas as pl
from jax.experimental.pallas import tpu as pltpu
from jax.experimental.pallas import tpu_sc as plsc

NQ = 4096
N = 100000
D = 128
NPAD = 100352
QB = 256
KB = 2048
BIN = 128
NBINS = NPAD // BIN
BPC = KB // BIN
NSEL = 16
NW = 32
RPW = NQ // NW


def _dist_body(q_ref, qsq_ref, kT_ref, ksq_ref, out_ref, bm_ref):
    q = q_ref[...]
    kT = kT_ref[...]
    dot = jnp.dot(q, kT, preferred_element_type=jnp.float32)
    d = qsq_ref[...] + ksq_ref[...] - 2.0 * dot
    out_ref[...] = d
    bm_ref[0] = jnp.min(d.reshape(QB, BPC, BIN), axis=2)


def _select_body(bm_ref, ids_ref):
    x = bm_ref[...]
    iota = jax.lax.broadcasted_iota(jnp.int32, (QB, NBINS), 1)
    ids_ref[...] = jnp.full((QB, NSEL), NBINS - 1, jnp.int32)
    for t in range(10):
        m = jnp.min(x, axis=1, keepdims=True)
        idx = jnp.min(jnp.where(x == m, iota, jnp.int32(2**30)),
                      axis=1, keepdims=True)
        ids_ref[:, t:t + 1] = idx
        x = jnp.where(iota == idx, jnp.inf, x)


def _topk_sc_body(dists_ref, ids_ref, outv_ref, outi_ref,
                  idsv, gath, outv_v, outi_v, sem):
    nc = 2
    wid = lax.axis_index("s") * nc + lax.axis_index("c")
    row0 = wid * RPW
    pltpu.sync_copy(ids_ref.at[pl.ds(row0 * NSEL, RPW * NSEL)], idsv)

    lane = lax.iota(jnp.int32, 16)

    def row_body(r, _):
        iv = idsv[pl.ds(r * NSEL, 16)]
        gx = iv + (row0 + r) * NBINS
        pltpu.async_copy(dists_ref.at[gx], gath, sem).wait()
        cand_v = jnp.full((16,), jnp.inf, jnp.float32)
        cand_i = jnp.zeros((16,), jnp.int32)
        for j in range(10):
            for v in range(BIN // 16):
                vals = gath[j, pl.ds(v * 16, 16)]
                lidx = (j * BIN + v * 16) + lane
                s_v, s_i = plsc.sort_key_val(vals, lidx)
                rv = lax.rev(s_v, (0,))
                ri = lax.rev(s_i, (0,))
                m_v = jnp.minimum(cand_v, rv)
                m_i = jnp.where(cand_v <= rv, cand_i, ri)
                cand_v, cand_i = plsc.sort_key_val(m_v, m_i)
        bsel = lax.gather(
            iv * BIN, lax.shift_right_logical(cand_i, 7)[:, None],
            lax.GatherDimensionNumbers(
                offset_dims=(), collapsed_slice_dims=(0,),
                start_index_map=(0,)),
            slice_sizes=(1,),
            mode=lax.GatherScatterMode.PROMISE_IN_BOUNDS)
        gidx = bsel + (cand_i & (BIN - 1))
        outv_v[pl.ds(r * 16, 16)] = cand_v
        outi_v[pl.ds(r * 16, 16)] = gidx
        return 0

    lax.fori_loop(0, RPW, row_body, 0)
    pltpu.sync_copy(outv_v, outv_ref.at[pl.ds(row0 * 16, RPW * 16)])
    pltpu.sync_copy(outi_v, outi_ref.at[pl.ds(row0 * 16, RPW * 16)])


def kernel(query, keys, W, k):
    q = query @ W.T
    qsq = jnp.sum(q * q, axis=1, keepdims=True)
    keysq = jnp.sum(keys * keys, axis=1)
    kT = jnp.concatenate(
        [keys.T, jnp.zeros((D, NPAD - N), jnp.float32)], axis=1)
    ksq_p = jnp.concatenate(
        [keysq, jnp.full((NPAD - N,), jnp.inf, jnp.float32)])
    ksq_p = ksq_p.reshape(1, NPAD)

    grid = (NPAD // KB, NQ // QB)
    dists, binmins = pl.pallas_call(
        _dist_body,
        grid=grid,
        in_specs=[
            pl.BlockSpec((QB, D), lambda kc, qb: (qb, 0)),
            pl.BlockSpec((QB, 1), lambda kc, qb: (qb, 0)),
            pl.BlockSpec((D, KB), lambda kc, qb: (0, kc)),
            pl.BlockSpec((1, KB), lambda kc, qb: (0, kc)),
        ],
        out_specs=[
            pl.BlockSpec((QB, KB), lambda kc, qb: (qb, kc)),
            pl.BlockSpec((1, QB, BPC), lambda kc, qb: (kc, qb, 0)),
        ],
        out_shape=[
            jax.ShapeDtypeStruct((NQ, NPAD), jnp.float32),
            jax.ShapeDtypeStruct((NPAD // KB, NQ, BPC), jnp.float32),
        ],
    )(q, qsq, kT, ksq_p)
    binmins = binmins.transpose(1, 0, 2).reshape(NQ, NBINS)

    ids = pl.pallas_call(
        _select_body,
        grid=(NQ // QB,),
        in_specs=[pl.BlockSpec((QB, NBINS), lambda qb: (qb, 0))],
        out_specs=pl.BlockSpec((QB, NSEL), lambda qb: (qb, 0)),
        out_shape=jax.ShapeDtypeStruct((NQ, NSEL), jnp.int32),
    )(binmins)

    sc = pl.kernel(
        _topk_sc_body,
        out_type=[
            jax.ShapeDtypeStruct((NQ * 16,), jnp.float32),
            jax.ShapeDtypeStruct((NQ * 16,), jnp.int32),
        ],
        mesh=plsc.VectorSubcoreMesh(core_axis_name="c", subcore_axis_name="s"),
        compiler_params=pltpu.CompilerParams(needs_layout_passes=False),
        scratch_types=[
            pltpu.VMEM((RPW * NSEL,), jnp.int32),
            pltpu.VMEM((NSEL, BIN), jnp.float32),
            pltpu.VMEM((RPW * 16,), jnp.float32),
            pltpu.VMEM((RPW * 16,), jnp.int32),
            pltpu.SemaphoreType.DMA,
        ],
    )
    outv, outi = sc(dists.reshape(NQ * NBINS, BIN), ids.reshape(NQ * NSEL))
    return (outv.reshape(NQ, 16)[:, :10], outi.reshape(NQ, 16)[:, :10])


# speedup vs baseline: 7.6640x; 1.6496x over previous
import functools

import jax
import jax.numpy as jnp
from jax import lax
from jax.experimental import pallas as pl
from jax.experimental.pallas import tpu as pltpu
from jax.experimental.pallas import tpu_sc as plsc

NQ = 4096
N = 100000
D = 128
NPAD = 100352
QB = 256
KB = 2048
BIN = 128
NBINS = NPAD // BIN
BPC = KB // BIN
NSEL = 16
NW = 32
RPW = NQ // NW


def _dist_body(q_ref, qsq_ref, kT_ref, ksq_ref, out_ref, bm_ref):
    q = q_ref[...]
    kT = kT_ref[...]
    dot = jnp.dot(q, kT, preferred_element_type=jnp.float32)
    d = qsq_ref[...] + ksq_ref[...] - 2.0 * dot
    d3 = d.reshape(QB, BPC, BIN)
    out_ref[...] = d3
    bm_ref[0] = jnp.min(d3, axis=2)


def _select_body(bm_ref, ids_ref):
    x = bm_ref[...]
    iota = jax.lax.broadcasted_iota(jnp.int32, (QB, NBINS), 1)
    ids_ref[...] = jnp.full((QB, NSEL), NBINS - 1, jnp.int32)
    for t in range(10):
        m = jnp.min(x, axis=1, keepdims=True)
        idx = jnp.min(jnp.where(x == m, iota, jnp.int32(2**30)),
                      axis=1, keepdims=True)
        ids_ref[:, t:t + 1] = idx
        x = jnp.where(iota == idx, jnp.inf, x)


def _topk_sc_body(dists_ref, ids_ref, outv_ref, outi_ref,
                  idsv, gath, outv_v, outi_v, sem):
    nc = 2
    wid = lax.axis_index("s") * nc + lax.axis_index("c")
    row0 = wid * RPW
    pltpu.sync_copy(ids_ref.at[pl.ds(row0 * NSEL, RPW * NSEL)], idsv)

    lane = lax.iota(jnp.int32, 16)

    def row_body(r, _):
        iv = idsv[pl.ds(r * NSEL, 16)]
        gx = iv + (row0 + r) * NBINS
        pltpu.async_copy(dists_ref.at[gx], gath, sem).wait()
        cand_v = jnp.full((16,), jnp.inf, jnp.float32)
        cand_i = jnp.zeros((16,), jnp.int32)
        for j in range(10):
            for v in range(BIN // 16):
                vals = gath[j, pl.ds(v * 16, 16)]
                lidx = (j * BIN + v * 16) + lane
                s_v, s_i = plsc.sort_key_val(vals, lidx)
                rv = lax.rev(s_v, (0,))
                ri = lax.rev(s_i, (0,))
                m_v = jnp.minimum(cand_v, rv)
                m_i = jnp.where(cand_v <= rv, cand_i, ri)
                cand_v, cand_i = plsc.sort_key_val(m_v, m_i)
        bsel = lax.gather(
            iv * BIN, lax.shift_right_logical(cand_i, 7)[:, None],
            lax.GatherDimensionNumbers(
                offset_dims=(), collapsed_slice_dims=(0,),
                start_index_map=(0,)),
            slice_sizes=(1,),
            mode=lax.GatherScatterMode.PROMISE_IN_BOUNDS)
        gidx = bsel + (cand_i & (BIN - 1))
        outv_v[pl.ds(r * 16, 16)] = cand_v
        outi_v[pl.ds(r * 16, 16)] = gidx
        return 0

    lax.fori_loop(0, RPW, row_body, 0)
    pltpu.sync_copy(outv_v, outv_ref.at[pl.ds(row0 * 16, RPW * 16)])
    pltpu.sync_copy(outi_v, outi_ref.at[pl.ds(row0 * 16, RPW * 16)])


def kernel(query, keys, W, k):
    q = query @ W.T
    qsq = jnp.sum(q * q, axis=1, keepdims=True)
    keysq = jnp.sum(keys * keys, axis=1)
    kT = jnp.concatenate(
        [keys.T, jnp.zeros((D, NPAD - N), jnp.float32)], axis=1)
    ksq_p = jnp.concatenate(
        [keysq, jnp.full((NPAD - N,), jnp.inf, jnp.float32)])
    ksq_p = ksq_p.reshape(1, NPAD)

    grid = (NPAD // KB, NQ // QB)
    dists, binmins = pl.pallas_call(
        _dist_body,
        grid=grid,
        in_specs=[
            pl.BlockSpec((QB, D), lambda kc, qb: (qb, 0)),
            pl.BlockSpec((QB, 1), lambda kc, qb: (qb, 0)),
            pl.BlockSpec((D, KB), lambda kc, qb: (0, kc)),
            pl.BlockSpec((1, KB), lambda kc, qb: (0, kc)),
        ],
        out_specs=[
            pl.BlockSpec((QB, BPC, BIN), lambda kc, qb: (qb, kc, 0)),
            pl.BlockSpec((1, QB, BPC), lambda kc, qb: (kc, qb, 0)),
        ],
        out_shape=[
            jax.ShapeDtypeStruct((NQ, NBINS, BIN), jnp.float32),
            jax.ShapeDtypeStruct((NPAD // KB, NQ, BPC), jnp.float32),
        ],
    )(q, qsq, kT, ksq_p)
    binmins = binmins.transpose(1, 0, 2).reshape(NQ, NBINS)

    ids = pl.pallas_call(
        _select_body,
        grid=(NQ // QB,),
        in_specs=[pl.BlockSpec((QB, NBINS), lambda qb: (qb, 0))],
        out_specs=pl.BlockSpec((QB, NSEL), lambda qb: (qb, 0)),
        out_shape=jax.ShapeDtypeStruct((NQ, NSEL), jnp.int32),
    )(binmins)

    sc = pl.kernel(
        _topk_sc_body,
        out_type=[
            jax.ShapeDtypeStruct((NQ * 16,), jnp.float32),
            jax.ShapeDtypeStruct((NQ * 16,), jnp.int32),
        ],
        mesh=plsc.VectorSubcoreMesh(core_axis_name="c", subcore_axis_name="s"),
        compiler_params=pltpu.CompilerParams(needs_layout_passes=False),
        scratch_types=[
            pltpu.VMEM((RPW * NSEL,), jnp.int32),
            pltpu.VMEM((NSEL, BIN), jnp.float32),
            pltpu.VMEM((RPW * 16,), jnp.float32),
            pltpu.VMEM((RPW * 16,), jnp.int32),
            pltpu.SemaphoreType.DMA,
        ],
    )
    outv, outi = sc(dists.reshape(NQ * NBINS, BIN), ids.reshape(NQ * NSEL))
    return (outv.reshape(NQ, 16)[:, :10], outi.reshape(NQ, 16)[:, :10])


# speedup vs baseline: 8.2877x; 1.0814x over previous
import functools

import jax
import jax.numpy as jnp
from jax import lax
from jax.experimental import pallas as pl
from jax.experimental.pallas import tpu as pltpu
from jax.experimental.pallas import tpu_sc as plsc

NQ = 4096
N = 100000
D = 128
NPAD = 100352
QB = 256
KB = 2048
KC = NPAD // KB
BIN = 128
NBINS = NPAD // BIN
BPC = KB // BIN
NSEL = 16
NW = 32
NCH = 2
QH = NQ // NCH
QBN = QH // QB
RPW = QH // NW


def _dist_body(q_ref, qsq_ref, kT_ref, ksq_ref, out_ref, ids_ref, bm_ref):
    kc = pl.program_id(0)
    qb = pl.program_id(1)
    q = q_ref[...]
    kT = kT_ref[...]
    dot = jnp.dot(q, kT, preferred_element_type=jnp.float32)
    d = qsq_ref[...] + ksq_ref[...] - 2.0 * dot
    d3 = d.reshape(QB, BPC, BIN)
    out_ref[...] = d3
    bm_ref[qb, pl.ds(kc * BPC, BPC), :] = jnp.transpose(jnp.min(d3, axis=2))

    @pl.when(kc == KC - 1)
    def _select():
        x = bm_ref[qb]
        iota = jax.lax.broadcasted_iota(jnp.int32, (NBINS, QB), 0)
        rows = []
        for t in range(10):
            m = jnp.min(x, axis=0, keepdims=True)
            idx = jnp.min(jnp.where(x == m, iota, jnp.int32(2**30)),
                          axis=0, keepdims=True)
            rows.append(idx)
            x = jnp.where(iota == idx, jnp.inf, x)
        rows.append(jnp.full((NSEL - 10, QB), NBINS - 1, jnp.int32))
        ids_ref[...] = jnp.transpose(jnp.concatenate(rows, axis=0))


def _topk_sc_body(dists_ref, ids_ref, outv_ref, outi_ref,
                  idsv, gath, outv_v, outi_v, sem):
    nc = 2
    wid = lax.axis_index("s") * nc + lax.axis_index("c")
    row0 = wid * RPW
    pltpu.sync_copy(ids_ref.at[pl.ds(row0 * NSEL, RPW * NSEL)], idsv)

    lane = lax.iota(jnp.int32, 16)

    def row_body(r, _):
        iv = idsv[pl.ds(r * NSEL, 16)]
        gx = iv + (row0 + r) * NBINS
        pltpu.async_copy(dists_ref.at[gx], gath, sem).wait()
        cand_v = jnp.full((16,), jnp.inf, jnp.float32)
        cand_i = jnp.zeros((16,), jnp.int32)
        for j in range(10):
            for v in range(BIN // 16):
                vals = gath[j, pl.ds(v * 16, 16)]
                lidx = (j * BIN + v * 16) + lane
                s_v, s_i = plsc.sort_key_val(vals, lidx)
                rv = lax.rev(s_v, (0,))
                ri = lax.rev(s_i, (0,))
                m_v = jnp.minimum(cand_v, rv)
                m_i = jnp.where(cand_v <= rv, cand_i, ri)
                cand_v, cand_i = plsc.sort_key_val(m_v, m_i)
        bsel = lax.gather(
            iv * BIN, lax.shift_right_logical(cand_i, 7)[:, None],
            lax.GatherDimensionNumbers(
                offset_dims=(), collapsed_slice_dims=(0,),
                start_index_map=(0,)),
            slice_sizes=(1,),
            mode=lax.GatherScatterMode.PROMISE_IN_BOUNDS)
        gidx = bsel + (cand_i & (BIN - 1))
        outv_v[pl.ds(r * 16, 16)] = cand_v
        outi_v[pl.ds(r * 16, 16)] = gidx
        return 0

    lax.fori_loop(0, RPW, row_body, 0)
    pltpu.sync_copy(outv_v, outv_ref.at[pl.ds(row0 * 16, RPW * 16)])
    pltpu.sync_copy(outi_v, outi_ref.at[pl.ds(row0 * 16, RPW * 16)])


def _phase1(q, qsq, kT, ksq_p):
    return pl.pallas_call(
        _dist_body,
        grid=(KC, QBN),
        in_specs=[
            pl.BlockSpec((QB, D), lambda kc, qb: (qb, 0)),
            pl.BlockSpec((QB, 1), lambda kc, qb: (qb, 0)),
            pl.BlockSpec((D, KB), lambda kc, qb: (0, kc)),
            pl.BlockSpec((1, KB), lambda kc, qb: (0, kc)),
        ],
        out_specs=[
            pl.BlockSpec((QB, BPC, BIN), lambda kc, qb: (qb, kc, 0)),
            pl.BlockSpec((QB, NSEL), lambda kc, qb: (qb, 0)),
        ],
        out_shape=[
            jax.ShapeDtypeStruct((QH, NBINS, BIN), jnp.float32),
            jax.ShapeDtypeStruct((QH, NSEL), jnp.int32),
        ],
        scratch_shapes=[pltpu.VMEM((QBN, NBINS, QB), jnp.float32)],
    )(q, qsq, kT, ksq_p)


def _phase2_sc(dists, ids):
    sc = pl.kernel(
        _topk_sc_body,
        out_type=[
            jax.ShapeDtypeStruct((QH * 16,), jnp.float32),
            jax.ShapeDtypeStruct((QH * 16,), jnp.int32),
        ],
        mesh=plsc.VectorSubcoreMesh(core_axis_name="c", subcore_axis_name="s"),
        compiler_params=pltpu.CompilerParams(needs_layout_passes=False),
        scratch_types=[
            pltpu.VMEM((RPW * NSEL,), jnp.int32),
            pltpu.VMEM((NSEL, BIN), jnp.float32),
            pltpu.VMEM((RPW * 16,), jnp.float32),
            pltpu.VMEM((RPW * 16,), jnp.int32),
            pltpu.SemaphoreType.DMA,
        ],
    )
    return sc(dists.reshape(QH * NBINS, BIN), ids.reshape(QH * NSEL))


def kernel(query, keys, W, k):
    q = query @ W.T
    qsq = jnp.sum(q * q, axis=1, keepdims=True)
    keysq = jnp.sum(keys * keys, axis=1)
    kT = jnp.concatenate(
        [keys.T, jnp.zeros((D, NPAD - N), jnp.float32)], axis=1)
    ksq_p = jnp.concatenate(
        [keysq, jnp.full((NPAD - N,), jnp.inf, jnp.float32)])
    ksq_p = ksq_p.reshape(1, NPAD)

    outs = []
    for c in range(NCH):
        qc = q[c * QH:(c + 1) * QH]
        qsqc = qsq[c * QH:(c + 1) * QH]
        dists, ids = _phase1(qc, qsqc, kT, ksq_p)
        outs.append(_phase2_sc(dists, ids))

    outv = jnp.concatenate([o[0].reshape(QH, 16) for o in outs], axis=0)
    outi = jnp.concatenate([o[1].reshape(QH, 16) for o in outs], axis=0)
    return (outv[:, :10], outi[:, :10])


# speedup vs baseline: 8.3730x; 1.0103x over previous
import functools

import jax
import jax.numpy as jnp
from jax import lax
from jax.experimental import pallas as pl
from jax.experimental.pallas import tpu as pltpu
from jax.experimental.pallas import tpu_sc as plsc

NQ = 4096
N = 100000
D = 128
NPAD = 100352
QB = 256
KB = 2048
KC = NPAD // KB
BIN = 128
NBINS = NPAD // BIN
BPC = KB // BIN
NSEL = 16
NW = 32
NCH = 4
QH = NQ // NCH
QBN = QH // QB
RPW = QH // NW


def _dist_body(q_ref, qsq_ref, kT_ref, ksq_ref, out_ref, ids_ref, bm_ref):
    kc = pl.program_id(0)
    qb = pl.program_id(1)
    q = q_ref[...]
    kT = kT_ref[...]
    dot = jnp.dot(q, kT, preferred_element_type=jnp.float32)
    d = qsq_ref[...] + ksq_ref[...] - 2.0 * dot
    d3 = d.reshape(QB, BPC, BIN)
    out_ref[...] = d3
    bm_ref[qb, pl.ds(kc * BPC, BPC), :] = jnp.transpose(jnp.min(d3, axis=2))

    @pl.when(kc == KC - 1)
    def _select():
        x = bm_ref[qb]
        iota = jax.lax.broadcasted_iota(jnp.int32, (NBINS, QB), 0)
        rows = []
        for t in range(10):
            m = jnp.min(x, axis=0, keepdims=True)
            idx = jnp.min(jnp.where(x == m, iota, jnp.int32(2**30)),
                          axis=0, keepdims=True)
            rows.append(idx)
            x = jnp.where(iota == idx, jnp.inf, x)
        rows.append(jnp.full((NSEL - 10, QB), NBINS - 1, jnp.int32))
        ids_ref[...] = jnp.transpose(jnp.concatenate(rows, axis=0))


def _topk_sc_body(dists_ref, ids_ref, outv_ref, outi_ref,
                  idsv, gath, outv_v, outi_v, sem):
    nc = 2
    wid = lax.axis_index("s") * nc + lax.axis_index("c")
    row0 = wid * RPW
    pltpu.sync_copy(ids_ref.at[pl.ds(row0 * NSEL, RPW * NSEL)], idsv)

    lane = lax.iota(jnp.int32, 16)

    def row_body(r, _):
        iv = idsv[pl.ds(r * NSEL, 16)]
        gx = iv + (row0 + r) * NBINS
        pltpu.async_copy(dists_ref.at[gx], gath, sem).wait()
        cand_v = jnp.full((16,), jnp.inf, jnp.float32)
        cand_i = jnp.zeros((16,), jnp.int32)
        for j in range(10):
            for v in range(BIN // 16):
                vals = gath[j, pl.ds(v * 16, 16)]
                lidx = (j * BIN + v * 16) + lane
                s_v, s_i = plsc.sort_key_val(vals, lidx)
                rv = lax.rev(s_v, (0,))
                ri = lax.rev(s_i, (0,))
                m_v = jnp.minimum(cand_v, rv)
                m_i = jnp.where(cand_v <= rv, cand_i, ri)
                cand_v, cand_i = plsc.sort_key_val(m_v, m_i)
        bsel = lax.gather(
            iv * BIN, lax.shift_right_logical(cand_i, 7)[:, None],
            lax.GatherDimensionNumbers(
                offset_dims=(), collapsed_slice_dims=(0,),
                start_index_map=(0,)),
            slice_sizes=(1,),
            mode=lax.GatherScatterMode.PROMISE_IN_BOUNDS)
        gidx = bsel + (cand_i & (BIN - 1))
        outv_v[pl.ds(r * 16, 16)] = cand_v
        outi_v[pl.ds(r * 16, 16)] = gidx
        return 0

    lax.fori_loop(0, RPW, row_body, 0)
    pltpu.sync_copy(outv_v, outv_ref.at[pl.ds(row0 * 16, RPW * 16)])
    pltpu.sync_copy(outi_v, outi_ref.at[pl.ds(row0 * 16, RPW * 16)])


def _phase1(q, qsq, kT, ksq_p):
    return pl.pallas_call(
        _dist_body,
        grid=(KC, QBN),
        in_specs=[
            pl.BlockSpec((QB, D), lambda kc, qb: (qb, 0)),
            pl.BlockSpec((QB, 1), lambda kc, qb: (qb, 0)),
            pl.BlockSpec((D, KB), lambda kc, qb: (0, kc)),
            pl.BlockSpec((1, KB), lambda kc, qb: (0, kc)),
        ],
        out_specs=[
            pl.BlockSpec((QB, BPC, BIN), lambda kc, qb: (qb, kc, 0)),
            pl.BlockSpec((QB, NSEL), lambda kc, qb: (qb, 0)),
        ],
        out_shape=[
            jax.ShapeDtypeStruct((QH, NBINS, BIN), jnp.float32),
            jax.ShapeDtypeStruct((QH, NSEL), jnp.int32),
        ],
        scratch_shapes=[pltpu.VMEM((QBN, NBINS, QB), jnp.float32)],
    )(q, qsq, kT, ksq_p)


def _phase2_sc(dists, ids):
    sc = pl.kernel(
        _topk_sc_body,
        out_type=[
            jax.ShapeDtypeStruct((QH * 16,), jnp.float32),
            jax.ShapeDtypeStruct((QH * 16,), jnp.int32),
        ],
        mesh=plsc.VectorSubcoreMesh(core_axis_name="c", subcore_axis_name="s"),
        compiler_params=pltpu.CompilerParams(needs_layout_passes=False),
        scratch_types=[
            pltpu.VMEM((RPW * NSEL,), jnp.int32),
            pltpu.VMEM((NSEL, BIN), jnp.float32),
            pltpu.VMEM((RPW * 16,), jnp.float32),
            pltpu.VMEM((RPW * 16,), jnp.int32),
            pltpu.SemaphoreType.DMA,
        ],
    )
    return sc(dists.reshape(QH * NBINS, BIN), ids.reshape(QH * NSEL))


def kernel(query, keys, W, k):
    q = query @ W.T
    qsq = jnp.sum(q * q, axis=1, keepdims=True)
    keysq = jnp.sum(keys * keys, axis=1)
    kT = jnp.concatenate(
        [keys.T, jnp.zeros((D, NPAD - N), jnp.float32)], axis=1)
    ksq_p = jnp.concatenate(
        [keysq, jnp.full((NPAD - N,), jnp.inf, jnp.float32)])
    ksq_p = ksq_p.reshape(1, NPAD)

    outs = []
    for c in range(NCH):
        qc = q[c * QH:(c + 1) * QH]
        qsqc = qsq[c * QH:(c + 1) * QH]
        dists, ids = _phase1(qc, qsqc, kT, ksq_p)
        outs.append(_phase2_sc(dists, ids))

    outv = jnp.concatenate([o[0].reshape(QH, 16) for o in outs], axis=0)
    outi = jnp.concatenate([o[1].reshape(QH, 16) for o in outs], axis=0)
    return (outv[:, :10], outi[:, :10])


# speedup vs baseline: 8.3748x; 1.0002x over previous
import functools

import jax
import jax.numpy as jnp
from jax import lax
from jax.experimental import pallas as pl
from jax.experimental.pallas import tpu as pltpu
from jax.experimental.pallas import tpu_sc as plsc

NQ = 4096
N = 100000
D = 128
NPAD = 100352
QB = 256
KB = 2048
KC = NPAD // KB
BIN = 128
NBINS = NPAD // BIN
BPC = KB // BIN
NSEL = 16
NW = 32
NCH = 4
QH = NQ // NCH
QBN = QH // QB
RPW = QH // NW


def _dist_body(q_ref, qsq_ref, kT_ref, ksq_ref, out_ref, ids_ref, bm_ref):
    kc = pl.program_id(0)
    qb = pl.program_id(1)
    q = q_ref[...]
    kT = kT_ref[...]
    dot = jnp.dot(q, kT, preferred_element_type=jnp.float32)
    d = qsq_ref[...] + ksq_ref[...] - 2.0 * dot
    d3 = d.reshape(QB, BPC, BIN)
    out_ref[...] = d3
    bm_ref[qb, pl.ds(kc * BPC, BPC), :] = jnp.transpose(jnp.min(d3, axis=2))

    @pl.when(kc == KC - 1)
    def _select():
        x = bm_ref[qb]
        iota = jax.lax.broadcasted_iota(jnp.int32, (NBINS, QB), 0)
        rows = []
        for t in range(10):
            m = jnp.min(x, axis=0, keepdims=True)
            idx = jnp.min(jnp.where(x == m, iota, jnp.int32(2**30)),
                          axis=0, keepdims=True)
            rows.append(idx)
            x = jnp.where(iota == idx, jnp.inf, x)
        rows.append(jnp.full((NSEL - 10, QB), NBINS - 1, jnp.int32))
        ids_ref[...] = jnp.transpose(jnp.concatenate(rows, axis=0))


def _topk_sc_body(dists_ref, ids_ref, outv_ref, outi_ref,
                  idsv, gath, outv_v, outi_v, sem):
    nc = 2
    wid = lax.axis_index("s") * nc + lax.axis_index("c")
    row0 = wid * RPW
    pltpu.sync_copy(ids_ref.at[pl.ds(row0 * NSEL, RPW * NSEL)], idsv)

    lane = lax.iota(jnp.int32, 16)

    def row_body(r, _):
        iv = idsv[pl.ds(r * NSEL, 16)]
        gx = iv + (row0 + r) * NBINS
        pltpu.async_copy(dists_ref.at[gx], gath, sem).wait()
        cand_v = jnp.full((16,), jnp.inf, jnp.float32)
        cand_i = jnp.zeros((16,), jnp.int32)
        for j in range(10):
            for v in range(BIN // 16):
                vals = gath[j, pl.ds(v * 16, 16)]
                lidx = (j * BIN + v * 16) + lane
                s_v, s_i = plsc.sort_key_val(vals, lidx)
                rv = lax.rev(s_v, (0,))
                ri = lax.rev(s_i, (0,))
                m_v = jnp.minimum(cand_v, rv)
                m_i = jnp.where(cand_v <= rv, cand_i, ri)
                cand_v, cand_i = plsc.sort_key_val(m_v, m_i)
        bsel = lax.gather(
            iv * BIN, lax.shift_right_logical(cand_i, 7)[:, None],
            lax.GatherDimensionNumbers(
                offset_dims=(), collapsed_slice_dims=(0,),
                start_index_map=(0,)),
            slice_sizes=(1,),
            mode=lax.GatherScatterMode.PROMISE_IN_BOUNDS)
        gidx = bsel + (cand_i & (BIN - 1))
        outv_v[pl.ds(r * 16, 16)] = cand_v
        outi_v[pl.ds(r * 16, 16)] = gidx
        return 0

    lax.fori_loop(0, RPW, row_body, 0)
    pltpu.sync_copy(outv_v, outv_ref.at[pl.ds(row0 * 16, RPW * 16)])
    pltpu.sync_copy(outi_v, outi_ref.at[pl.ds(row0 * 16, RPW * 16)])


def _phase1(q, qsq, kT, ksq_p):
    return pl.pallas_call(
        _dist_body,
        grid=(KC, QBN),
        in_specs=[
            pl.BlockSpec((QB, D), lambda kc, qb: (qb, 0)),
            pl.BlockSpec((QB, 1), lambda kc, qb: (qb, 0)),
            pl.BlockSpec((D, KB), lambda kc, qb: (0, kc)),
            pl.BlockSpec((1, KB), lambda kc, qb: (0, kc)),
        ],
        out_specs=[
            pl.BlockSpec((QB, BPC, BIN), lambda kc, qb: (qb, kc, 0)),
            pl.BlockSpec((QB, NSEL), lambda kc, qb: (qb, 0)),
        ],
        out_shape=[
            jax.ShapeDtypeStruct((QH, NBINS, BIN), jnp.float32),
            jax.ShapeDtypeStruct((QH, NSEL), jnp.int32),
        ],
        scratch_shapes=[pltpu.VMEM((QBN, NBINS, QB), jnp.float32)],
        compiler_params=pltpu.CompilerParams(
            dimension_semantics=("arbitrary", "parallel")),
    )(q, qsq, kT, ksq_p)


def _phase2_sc(dists, ids):
    sc = pl.kernel(
        _topk_sc_body,
        out_type=[
            jax.ShapeDtypeStruct((QH * 16,), jnp.float32),
            jax.ShapeDtypeStruct((QH * 16,), jnp.int32),
        ],
        mesh=plsc.VectorSubcoreMesh(core_axis_name="c", subcore_axis_name="s"),
        compiler_params=pltpu.CompilerParams(needs_layout_passes=False),
        scratch_types=[
            pltpu.VMEM((RPW * NSEL,), jnp.int32),
            pltpu.VMEM((NSEL, BIN), jnp.float32),
            pltpu.VMEM((RPW * 16,), jnp.float32),
            pltpu.VMEM((RPW * 16,), jnp.int32),
            pltpu.SemaphoreType.DMA,
        ],
    )
    return sc(dists.reshape(QH * NBINS, BIN), ids.reshape(QH * NSEL))


def kernel(query, keys, W, k):
    q = query @ W.T
    qsq = jnp.sum(q * q, axis=1, keepdims=True)
    keysq = jnp.sum(keys * keys, axis=1)
    kT = jnp.concatenate(
        [keys.T, jnp.zeros((D, NPAD - N), jnp.float32)], axis=1)
    ksq_p = jnp.concatenate(
        [keysq, jnp.full((NPAD - N,), jnp.inf, jnp.float32)])
    ksq_p = ksq_p.reshape(1, NPAD)

    outs = []
    for c in range(NCH):
        qc = q[c * QH:(c + 1) * QH]
        qsqc = qsq[c * QH:(c + 1) * QH]
        dists, ids = _phase1(qc, qsqc, kT, ksq_p)
        outs.append(_phase2_sc(dists, ids))

    outv = jnp.concatenate([o[0].reshape(QH, 16) for o in outs], axis=0)
    outi = jnp.concatenate([o[1].reshape(QH, 16) for o in outs], axis=0)
    return (outv[:, :10], outi[:, :10])


# speedup vs baseline: 8.4769x; 1.0122x over previous
import functools

import jax
import jax.numpy as jnp
from jax import lax
from jax.experimental import pallas as pl
from jax.experimental.pallas import tpu as pltpu
from jax.experimental.pallas import tpu_sc as plsc

NQ = 4096
N = 100000
D = 128
NPAD = 100352
QB = 256
KB = 2048
KC = NPAD // KB
BIN = 128
NBINS = NPAD // BIN
BPC = KB // BIN
NSEL = 16
NW = 32
NCH = 4
QH = NQ // NCH
QBN = QH // QB
RPW = QH // NW


def _dist_body(q_ref, qsq_ref, kT_ref, ksq_ref, out_ref, ids_ref, bm_ref):
    kc = pl.program_id(0)
    qb = pl.program_id(1)
    q = q_ref[...]
    kT = kT_ref[...]
    dot = jnp.dot(q, kT, preferred_element_type=jnp.float32)
    d = qsq_ref[...] + ksq_ref[...] - 2.0 * dot
    d3 = d.reshape(QB, BPC, BIN)
    out_ref[...] = d3.reshape(1, QB, BPC, BIN)
    bm_ref[qb, pl.ds(kc * BPC, BPC), :] = jnp.transpose(jnp.min(d3, axis=2))

    @pl.when(kc == KC - 1)
    def _select():
        x = bm_ref[qb]
        iota = jax.lax.broadcasted_iota(jnp.int32, (NBINS, QB), 0)
        rows = []
        for t in range(10):
            m = jnp.min(x, axis=0, keepdims=True)
            idx = jnp.min(jnp.where(x == m, iota, jnp.int32(2**30)),
                          axis=0, keepdims=True)
            rows.append(idx)
            x = jnp.where(iota == idx, jnp.inf, x)
        rows.append(jnp.full((NSEL - 10, QB), NBINS - 1, jnp.int32))
        ids_ref[...] = jnp.transpose(jnp.concatenate(rows, axis=0))


def _topk_sc_body(dists_ref, ids_ref, outv_ref, outi_ref,
                  idsv, gath, outv_v, outi_v, sem):
    nc = 2
    wid = lax.axis_index("s") * nc + lax.axis_index("c")
    row0 = wid * RPW
    pltpu.sync_copy(ids_ref.at[pl.ds(row0 * NSEL, RPW * NSEL)], idsv)

    lane = lax.iota(jnp.int32, 16)

    def row_body(r, _):
        iv = idsv[pl.ds(r * NSEL, 16)]
        gx = (lax.shift_right_logical(iv, 4) * (QH * BPC)
              + (row0 + r) * BPC + (iv & (BPC - 1)))
        pltpu.async_copy(dists_ref.at[gx], gath, sem).wait()
        cand_v = jnp.full((16,), jnp.inf, jnp.float32)
        cand_i = jnp.zeros((16,), jnp.int32)
        for j in range(10):
            for v in range(BIN // 16):
                vals = gath[j, pl.ds(v * 16, 16)]
                lidx = (j * BIN + v * 16) + lane
                s_v, s_i = plsc.sort_key_val(vals, lidx)
                rv = lax.rev(s_v, (0,))
                ri = lax.rev(s_i, (0,))
                m_v = jnp.minimum(cand_v, rv)
                m_i = jnp.where(cand_v <= rv, cand_i, ri)
                cand_v, cand_i = plsc.sort_key_val(m_v, m_i)
        bsel = lax.gather(
            iv * BIN, lax.shift_right_logical(cand_i, 7)[:, None],
            lax.GatherDimensionNumbers(
                offset_dims=(), collapsed_slice_dims=(0,),
                start_index_map=(0,)),
            slice_sizes=(1,),
            mode=lax.GatherScatterMode.PROMISE_IN_BOUNDS)
        gidx = bsel + (cand_i & (BIN - 1))
        outv_v[pl.ds(r * 16, 16)] = cand_v
        outi_v[pl.ds(r * 16, 16)] = gidx
        return 0

    lax.fori_loop(0, RPW, row_body, 0)
    pltpu.sync_copy(outv_v, outv_ref.at[pl.ds(row0 * 16, RPW * 16)])
    pltpu.sync_copy(outi_v, outi_ref.at[pl.ds(row0 * 16, RPW * 16)])


def _phase1(q, qsq, kT, ksq_p):
    return pl.pallas_call(
        _dist_body,
        grid=(KC, QBN),
        in_specs=[
            pl.BlockSpec((QB, D), lambda kc, qb: (qb, 0)),
            pl.BlockSpec((QB, 1), lambda kc, qb: (qb, 0)),
            pl.BlockSpec((D, KB), lambda kc, qb: (0, kc)),
            pl.BlockSpec((1, KB), lambda kc, qb: (0, kc)),
        ],
        out_specs=[
            pl.BlockSpec((1, QB, BPC, BIN), lambda kc, qb: (kc, qb, 0, 0)),
            pl.BlockSpec((QB, NSEL), lambda kc, qb: (qb, 0)),
        ],
        out_shape=[
            jax.ShapeDtypeStruct((KC, QH, BPC, BIN), jnp.float32),
            jax.ShapeDtypeStruct((QH, NSEL), jnp.int32),
        ],
        scratch_shapes=[pltpu.VMEM((QBN, NBINS, QB), jnp.float32)],
        compiler_params=pltpu.CompilerParams(
            dimension_semantics=("arbitrary", "parallel")),
    )(q, qsq, kT, ksq_p)


def _phase2_sc(dists, ids):
    sc = pl.kernel(
        _topk_sc_body,
        out_type=[
            jax.ShapeDtypeStruct((QH * 16,), jnp.float32),
            jax.ShapeDtypeStruct((QH * 16,), jnp.int32),
        ],
        mesh=plsc.VectorSubcoreMesh(core_axis_name="c", subcore_axis_name="s"),
        compiler_params=pltpu.CompilerParams(needs_layout_passes=False),
        scratch_types=[
            pltpu.VMEM((RPW * NSEL,), jnp.int32),
            pltpu.VMEM((NSEL, BIN), jnp.float32),
            pltpu.VMEM((RPW * 16,), jnp.float32),
            pltpu.VMEM((RPW * 16,), jnp.int32),
            pltpu.SemaphoreType.DMA,
        ],
    )
    return sc(dists.reshape(KC * QH * BPC, BIN), ids.reshape(QH * NSEL))


def kernel(query, keys, W, k):
    q = query @ W.T
    qsq = jnp.sum(q * q, axis=1, keepdims=True)
    keysq = jnp.sum(keys * keys, axis=1)
    kT = jnp.concatenate(
        [keys.T, jnp.zeros((D, NPAD - N), jnp.float32)], axis=1)
    ksq_p = jnp.concatenate(
        [keysq, jnp.full((NPAD - N,), jnp.inf, jnp.float32)])
    ksq_p = ksq_p.reshape(1, NPAD)

    outs = []
    for c in range(NCH):
        qc = q[c * QH:(c + 1) * QH]
        qsqc = qsq[c * QH:(c + 1) * QH]
        dists, ids = _phase1(qc, qsqc, kT, ksq_p)
        outs.append(_phase2_sc(dists, ids))

    outv = jnp.concatenate([o[0].reshape(QH, 16) for o in outs], axis=0)
    outi = jnp.concatenate([o[1].reshape(QH, 16) for o in outs], axis=0)
    return (outv[:, :10], outi[:, :10])


# speedup vs baseline: 10.0049x; 1.1803x over previous
import functools

import jax
import jax.numpy as jnp
from jax import lax
from jax.experimental import pallas as pl
from jax.experimental.pallas import tpu as pltpu
from jax.experimental.pallas import tpu_sc as plsc

NQ = 4096
N = 100000
D = 128
NPAD = 100352
QB = 512
KB = 2048
KC = NPAD // KB
BIN = 128
NBINS = NPAD // BIN
BPC = KB // BIN
NSEL = 16
NW = 32
NCH = 4
QH = NQ // NCH
QBN = QH // QB
RPW = QH // NW


def _dist_body(q_ref, qsq_ref, kT_ref, ksq_ref, out_ref, ids_ref, bm_ref):
    kc = pl.program_id(0)
    qb = pl.program_id(1)
    q = q_ref[...]
    kT = kT_ref[...]
    dot = jnp.dot(q, kT, preferred_element_type=jnp.float32)
    d = qsq_ref[...] + ksq_ref[...] + dot
    d3 = d.reshape(QB, BPC, BIN)
    out_ref[...] = d3.reshape(1, QB, BPC, BIN)
    bm_ref[qb, pl.ds(kc * BPC, BPC), :] = jnp.transpose(jnp.min(d3, axis=2))

    @pl.when(kc == KC - 1)
    def _select():
        x = bm_ref[qb]
        iota = jax.lax.broadcasted_iota(jnp.int32, (NBINS, QB), 0)
        rows = []
        for t in range(10):
            m = jnp.min(x, axis=0, keepdims=True)
            idx = jnp.min(jnp.where(x == m, iota, jnp.int32(2**30)),
                          axis=0, keepdims=True)
            rows.append(idx)
            x = jnp.where(iota == idx, jnp.inf, x)
        rows.append(jnp.full((NSEL - 10, QB), NBINS - 1, jnp.int32))
        ids_ref[...] = jnp.transpose(jnp.concatenate(rows, axis=0))


def _topk_sc_body(dists_ref, ids_ref, outv_ref, outi_ref,
                  idsv, gath, outv_v, outi_v, sem):
    nc = 2
    wid = lax.axis_index("s") * nc + lax.axis_index("c")
    row0 = wid * RPW
    pltpu.sync_copy(ids_ref.at[pl.ds(row0 * NSEL, RPW * NSEL)], idsv)

    lane = lax.iota(jnp.int32, 16)

    def row_body(r, _):
        iv = idsv[pl.ds(r * NSEL, 16)]
        gx = (lax.shift_right_logical(iv, 4) * (QH * BPC)
              + (row0 + r) * BPC + (iv & (BPC - 1)))
        pltpu.async_copy(dists_ref.at[gx], gath, sem).wait()
        cand_v = jnp.full((16,), jnp.inf, jnp.float32)
        cand_i = jnp.zeros((16,), jnp.int32)
        for j in range(10):
            for v in range(BIN // 16):
                vals = gath[j, pl.ds(v * 16, 16)]
                lidx = (j * BIN + v * 16) + lane
                s_v, s_i = plsc.sort_key_val(vals, lidx)
                rv = lax.rev(s_v, (0,))
                ri = lax.rev(s_i, (0,))
                m_v = jnp.minimum(cand_v, rv)
                m_i = jnp.where(cand_v <= rv, cand_i, ri)
                cand_v, cand_i = plsc.sort_key_val(m_v, m_i)
        bsel = lax.gather(
            iv * BIN, lax.shift_right_logical(cand_i, 7)[:, None],
            lax.GatherDimensionNumbers(
                offset_dims=(), collapsed_slice_dims=(0,),
                start_index_map=(0,)),
            slice_sizes=(1,),
            mode=lax.GatherScatterMode.PROMISE_IN_BOUNDS)
        gidx = bsel + (cand_i & (BIN - 1))
        outv_v[pl.ds(r * 16, 16)] = cand_v
        outi_v[pl.ds(r * 16, 16)] = gidx
        return 0

    lax.fori_loop(0, RPW, row_body, 0)
    pltpu.sync_copy(outv_v, outv_ref.at[pl.ds(row0 * 16, RPW * 16)])
    pltpu.sync_copy(outi_v, outi_ref.at[pl.ds(row0 * 16, RPW * 16)])


def _phase1(q, qsq, kT, ksq_p):
    return pl.pallas_call(
        _dist_body,
        grid=(KC, QBN),
        in_specs=[
            pl.BlockSpec((QB, D), lambda kc, qb: (qb, 0)),
            pl.BlockSpec((QB, 1), lambda kc, qb: (qb, 0)),
            pl.BlockSpec((D, KB), lambda kc, qb: (0, kc)),
            pl.BlockSpec((1, KB), lambda kc, qb: (0, kc)),
        ],
        out_specs=[
            pl.BlockSpec((1, QB, BPC, BIN), lambda kc, qb: (kc, qb, 0, 0)),
            pl.BlockSpec((QB, NSEL), lambda kc, qb: (qb, 0)),
        ],
        out_shape=[
            jax.ShapeDtypeStruct((KC, QH, BPC, BIN), jnp.float32),
            jax.ShapeDtypeStruct((QH, NSEL), jnp.int32),
        ],
        scratch_shapes=[pltpu.VMEM((QBN, NBINS, QB), jnp.float32)],
        compiler_params=pltpu.CompilerParams(
            dimension_semantics=("arbitrary", "parallel")),
    )(q, qsq, kT, ksq_p)


def _phase2_sc(dists, ids):
    sc = pl.kernel(
        _topk_sc_body,
        out_type=[
            jax.ShapeDtypeStruct((QH * 16,), jnp.float32),
            jax.ShapeDtypeStruct((QH * 16,), jnp.int32),
        ],
        mesh=plsc.VectorSubcoreMesh(core_axis_name="c", subcore_axis_name="s"),
        compiler_params=pltpu.CompilerParams(needs_layout_passes=False),
        scratch_types=[
            pltpu.VMEM((RPW * NSEL,), jnp.int32),
            pltpu.VMEM((NSEL, BIN), jnp.float32),
            pltpu.VMEM((RPW * 16,), jnp.float32),
            pltpu.VMEM((RPW * 16,), jnp.int32),
            pltpu.SemaphoreType.DMA,
        ],
    )
    return sc(dists.reshape(KC * QH * BPC, BIN), ids.reshape(QH * NSEL))


def kernel(query, keys, W, k):
    q = query @ W.T
    qsq = jnp.sum(q * q, axis=1, keepdims=True)
    q2 = q * jnp.float32(-2.0)
    keysq = jnp.sum(keys * keys, axis=1)
    kT = jnp.concatenate(
        [keys.T, jnp.zeros((D, NPAD - N), jnp.float32)], axis=1)
    ksq_p = jnp.concatenate(
        [keysq, jnp.full((NPAD - N,), jnp.inf, jnp.float32)])
    ksq_p = ksq_p.reshape(1, NPAD)

    outs = []
    for c in range(NCH):
        qc = q2[c * QH:(c + 1) * QH]
        qsqc = qsq[c * QH:(c + 1) * QH]
        dists, ids = _phase1(qc, qsqc, kT, ksq_p)
        outs.append(_phase2_sc(dists, ids))

    outv = jnp.concatenate([o[0].reshape(QH, 16) for o in outs], axis=0)
    outi = jnp.concatenate([o[1].reshape(QH, 16) for o in outs], axis=0)
    return (outv[:, :10], outi[:, :10])


# speedup vs baseline: 11.2697x; 1.1264x over previous
import functools

import jax
import jax.numpy as jnp
from jax import lax
from jax.experimental import pallas as pl
from jax.experimental.pallas import tpu as pltpu
from jax.experimental.pallas import tpu_sc as plsc

NQ = 4096
N = 100000
D = 128
NPAD = 100352
QB = 1024
KB = 2048
KC = NPAD // KB
BIN = 128
NBINS = NPAD // BIN
BPC = KB // BIN
NSEL = 16
NW = 32
NCH = 4
QH = NQ // NCH
QBN = QH // QB
RPW = QH // NW


def _dist_body(q_ref, qsq_ref, kT_ref, ksq_ref, out_ref, ids_ref, bm_ref):
    kc = pl.program_id(0)
    qb = pl.program_id(1)
    q = q_ref[...]
    kT = kT_ref[...]
    dot = jnp.dot(q, kT, preferred_element_type=jnp.float32)
    d = qsq_ref[...] + ksq_ref[...] + dot
    d3 = d.reshape(QB, BPC, BIN)
    out_ref[...] = d3.reshape(1, QB, BPC, BIN)
    bm_ref[qb, pl.ds(kc * BPC, BPC), :] = jnp.transpose(jnp.min(d3, axis=2))

    @pl.when(kc == KC - 1)
    def _select():
        x = bm_ref[qb]
        iota = jax.lax.broadcasted_iota(jnp.int32, (NBINS, QB), 0)
        rows = []
        for t in range(10):
            m = jnp.min(x, axis=0, keepdims=True)
            idx = jnp.min(jnp.where(x == m, iota, jnp.int32(2**30)),
                          axis=0, keepdims=True)
            rows.append(idx)
            x = jnp.where(iota == idx, jnp.inf, x)
        rows.append(jnp.full((NSEL - 10, QB), NBINS - 1, jnp.int32))
        ids_ref[...] = jnp.transpose(jnp.concatenate(rows, axis=0))


def _topk_sc_body(dists_ref, ids_ref, outv_ref, outi_ref,
                  idsv, gath, outv_v, outi_v, sem):
    nc = 2
    wid = lax.axis_index("s") * nc + lax.axis_index("c")
    row0 = wid * RPW
    pltpu.sync_copy(ids_ref.at[pl.ds(row0 * NSEL, RPW * NSEL)], idsv)

    lane = lax.iota(jnp.int32, 16)

    def row_body(r, _):
        iv = idsv[pl.ds(r * NSEL, 16)]
        gx = (lax.shift_right_logical(iv, 4) * (QH * BPC)
              + (row0 + r) * BPC + (iv & (BPC - 1)))
        pltpu.async_copy(dists_ref.at[gx], gath, sem).wait()
        cand_v = jnp.full((16,), jnp.inf, jnp.float32)
        cand_i = jnp.zeros((16,), jnp.int32)
        for j in range(10):
            for v in range(BIN // 16):
                vals = gath[j, pl.ds(v * 16, 16)]
                lidx = (j * BIN + v * 16) + lane
                s_v, s_i = plsc.sort_key_val(vals, lidx)
                rv = lax.rev(s_v, (0,))
                ri = lax.rev(s_i, (0,))
                m_v = jnp.minimum(cand_v, rv)
                m_i = jnp.where(cand_v <= rv, cand_i, ri)
                cand_v, cand_i = plsc.sort_key_val(m_v, m_i)
        bsel = lax.gather(
            iv * BIN, lax.shift_right_logical(cand_i, 7)[:, None],
            lax.GatherDimensionNumbers(
                offset_dims=(), collapsed_slice_dims=(0,),
                start_index_map=(0,)),
            slice_sizes=(1,),
            mode=lax.GatherScatterMode.PROMISE_IN_BOUNDS)
        gidx = bsel + (cand_i & (BIN - 1))
        outv_v[pl.ds(r * 16, 16)] = cand_v
        outi_v[pl.ds(r * 16, 16)] = gidx
        return 0

    lax.fori_loop(0, RPW, row_body, 0)
    pltpu.sync_copy(outv_v, outv_ref.at[pl.ds(row0 * 16, RPW * 16)])
    pltpu.sync_copy(outi_v, outi_ref.at[pl.ds(row0 * 16, RPW * 16)])


def _phase1(q, qsq, kT, ksq_p):
    return pl.pallas_call(
        _dist_body,
        grid=(KC, QBN),
        in_specs=[
            pl.BlockSpec((QB, D), lambda kc, qb: (qb, 0)),
            pl.BlockSpec((QB, 1), lambda kc, qb: (qb, 0)),
            pl.BlockSpec((D, KB), lambda kc, qb: (0, kc)),
            pl.BlockSpec((1, KB), lambda kc, qb: (0, kc)),
        ],
        out_specs=[
            pl.BlockSpec((1, QB, BPC, BIN), lambda kc, qb: (kc, qb, 0, 0)),
            pl.BlockSpec((QB, NSEL), lambda kc, qb: (qb, 0)),
        ],
        out_shape=[
            jax.ShapeDtypeStruct((KC, QH, BPC, BIN), jnp.float32),
            jax.ShapeDtypeStruct((QH, NSEL), jnp.int32),
        ],
        scratch_shapes=[pltpu.VMEM((QBN, NBINS, QB), jnp.float32)],
        compiler_params=pltpu.CompilerParams(
            dimension_semantics=("arbitrary", "parallel")),
    )(q, qsq, kT, ksq_p)


def _phase2_sc(dists, ids):
    sc = pl.kernel(
        _topk_sc_body,
        out_type=[
            jax.ShapeDtypeStruct((QH * 16,), jnp.float32),
            jax.ShapeDtypeStruct((QH * 16,), jnp.int32),
        ],
        mesh=plsc.VectorSubcoreMesh(core_axis_name="c", subcore_axis_name="s"),
        compiler_params=pltpu.CompilerParams(needs_layout_passes=False),
        scratch_types=[
            pltpu.VMEM((RPW * NSEL,), jnp.int32),
            pltpu.VMEM((NSEL, BIN), jnp.float32),
            pltpu.VMEM((RPW * 16,), jnp.float32),
            pltpu.VMEM((RPW * 16,), jnp.int32),
            pltpu.SemaphoreType.DMA,
        ],
    )
    return sc(dists.reshape(KC * QH * BPC, BIN), ids.reshape(QH * NSEL))


def kernel(query, keys, W, k):
    q = query @ W.T
    qsq = jnp.sum(q * q, axis=1, keepdims=True)
    q2 = q * jnp.float32(-2.0)
    keysq = jnp.sum(keys * keys, axis=1)
    kT = jnp.concatenate(
        [keys.T, jnp.zeros((D, NPAD - N), jnp.float32)], axis=1)
    ksq_p = jnp.concatenate(
        [keysq, jnp.full((NPAD - N,), jnp.inf, jnp.float32)])
    ksq_p = ksq_p.reshape(1, NPAD)

    outs = []
    for c in range(NCH):
        qc = q2[c * QH:(c + 1) * QH]
        qsqc = qsq[c * QH:(c + 1) * QH]
        dists, ids = _phase1(qc, qsqc, kT, ksq_p)
        outs.append(_phase2_sc(dists, ids))

    outv = jnp.concatenate([o[0].reshape(QH, 16) for o in outs], axis=0)
    outi = jnp.concatenate([o[1].reshape(QH, 16) for o in outs], axis=0)
    return (outv[:, :10], outi[:, :10])


# speedup vs baseline: 11.5796x; 1.0275x over previous
import functools

import jax
import jax.numpy as jnp
from jax import lax
from jax.experimental import pallas as pl
from jax.experimental.pallas import tpu as pltpu
from jax.experimental.pallas import tpu_sc as plsc

NQ = 4096
N = 100000
D = 128
NPAD = 102400
QB = 1024
KB = 4096
KC = NPAD // KB
BIN = 128
NBINS = NPAD // BIN
BPC = KB // BIN
NSEL = 16
NW = 32
NCH = 4
QH = NQ // NCH
QBN = QH // QB
RPW = QH // NW


def _dist_body(q_ref, qsq_ref, kT_ref, ksq_ref, out_ref, ids_ref, bm_ref):
    kc = pl.program_id(0)
    qb = pl.program_id(1)
    q = q_ref[...]
    kT = kT_ref[...]
    dot = jnp.dot(q, kT, preferred_element_type=jnp.float32)
    d = qsq_ref[...] + ksq_ref[...] + dot
    d3 = d.reshape(QB, BPC, BIN)
    out_ref[...] = d3.reshape(1, QB, BPC, BIN)
    bm_ref[qb, pl.ds(kc * BPC, BPC), :] = jnp.transpose(jnp.min(d3, axis=2))

    @pl.when(kc == KC - 1)
    def _select():
        x = bm_ref[qb]
        iota = jax.lax.broadcasted_iota(jnp.int32, (NBINS, QB), 0)
        rows = []
        for t in range(10):
            m = jnp.min(x, axis=0, keepdims=True)
            idx = jnp.min(jnp.where(x == m, iota, jnp.int32(2**30)),
                          axis=0, keepdims=True)
            rows.append(idx)
            x = jnp.where(iota == idx, jnp.inf, x)
        rows.append(jnp.full((NSEL - 10, QB), NBINS - 1, jnp.int32))
        ids_ref[...] = jnp.transpose(jnp.concatenate(rows, axis=0))


def _topk_sc_body(dists_ref, ids_ref, outv_ref, outi_ref,
                  idsv, gath, outv_v, outi_v, sem):
    nc = 2
    wid = lax.axis_index("s") * nc + lax.axis_index("c")
    row0 = wid * RPW
    pltpu.sync_copy(ids_ref.at[pl.ds(row0 * NSEL, RPW * NSEL)], idsv)

    lane = lax.iota(jnp.int32, 16)

    def row_body(r, _):
        iv = idsv[pl.ds(r * NSEL, 16)]
        gx = (lax.shift_right_logical(iv, BPC.bit_length() - 1) * (QH * BPC)
              + (row0 + r) * BPC + (iv & (BPC - 1)))
        pltpu.async_copy(dists_ref.at[gx], gath, sem).wait()
        cand_v = jnp.full((16,), jnp.inf, jnp.float32)
        cand_i = jnp.zeros((16,), jnp.int32)
        for j in range(10):
            for v in range(BIN // 16):
                vals = gath[j, pl.ds(v * 16, 16)]
                lidx = (j * BIN + v * 16) + lane
                s_v, s_i = plsc.sort_key_val(vals, lidx)
                rv = lax.rev(s_v, (0,))
                ri = lax.rev(s_i, (0,))
                m_v = jnp.minimum(cand_v, rv)
                m_i = jnp.where(cand_v <= rv, cand_i, ri)
                cand_v, cand_i = plsc.sort_key_val(m_v, m_i)
        bsel = lax.gather(
            iv * BIN, lax.shift_right_logical(cand_i, 7)[:, None],
            lax.GatherDimensionNumbers(
                offset_dims=(), collapsed_slice_dims=(0,),
                start_index_map=(0,)),
            slice_sizes=(1,),
            mode=lax.GatherScatterMode.PROMISE_IN_BOUNDS)
        gidx = bsel + (cand_i & (BIN - 1))
        outv_v[pl.ds(r * 16, 16)] = cand_v
        outi_v[pl.ds(r * 16, 16)] = gidx
        return 0

    lax.fori_loop(0, RPW, row_body, 0)
    pltpu.sync_copy(outv_v, outv_ref.at[pl.ds(row0 * 16, RPW * 16)])
    pltpu.sync_copy(outi_v, outi_ref.at[pl.ds(row0 * 16, RPW * 16)])


def _phase1(q, qsq, kT, ksq_p):
    return pl.pallas_call(
        _dist_body,
        grid=(KC, QBN),
        in_specs=[
            pl.BlockSpec((QB, D), lambda kc, qb: (qb, 0)),
            pl.BlockSpec((QB, 1), lambda kc, qb: (qb, 0)),
            pl.BlockSpec((D, KB), lambda kc, qb: (0, kc)),
            pl.BlockSpec((1, KB), lambda kc, qb: (0, kc)),
        ],
        out_specs=[
            pl.BlockSpec((1, QB, BPC, BIN), lambda kc, qb: (kc, qb, 0, 0)),
            pl.BlockSpec((QB, NSEL), lambda kc, qb: (qb, 0)),
        ],
        out_shape=[
            jax.ShapeDtypeStruct((KC, QH, BPC, BIN), jnp.float32),
            jax.ShapeDtypeStruct((QH, NSEL), jnp.int32),
        ],
        scratch_shapes=[pltpu.VMEM((QBN, NBINS, QB), jnp.float32)],
        compiler_params=pltpu.CompilerParams(
            dimension_semantics=("arbitrary", "parallel")),
    )(q, qsq, kT, ksq_p)


def _phase2_sc(dists, ids):
    sc = pl.kernel(
        _topk_sc_body,
        out_type=[
            jax.ShapeDtypeStruct((QH * 16,), jnp.float32),
            jax.ShapeDtypeStruct((QH * 16,), jnp.int32),
        ],
        mesh=plsc.VectorSubcoreMesh(core_axis_name="c", subcore_axis_name="s"),
        compiler_params=pltpu.CompilerParams(needs_layout_passes=False),
        scratch_types=[
            pltpu.VMEM((RPW * NSEL,), jnp.int32),
            pltpu.VMEM((NSEL, BIN), jnp.float32),
            pltpu.VMEM((RPW * 16,), jnp.float32),
            pltpu.VMEM((RPW * 16,), jnp.int32),
            pltpu.SemaphoreType.DMA,
        ],
    )
    return sc(dists.reshape(KC * QH * BPC, BIN), ids.reshape(QH * NSEL))


def kernel(query, keys, W, k):
    q = query @ W.T
    qsq = jnp.sum(q * q, axis=1, keepdims=True)
    q2 = q * jnp.float32(-2.0)
    keysq = jnp.sum(keys * keys, axis=1)
    kT = jnp.concatenate(
        [keys.T, jnp.zeros((D, NPAD - N), jnp.float32)], axis=1)
    ksq_p = jnp.concatenate(
        [keysq, jnp.full((NPAD - N,), jnp.inf, jnp.float32)])
    ksq_p = ksq_p.reshape(1, NPAD)

    outs = []
    for c in range(NCH):
        qc = q2[c * QH:(c + 1) * QH]
        qsqc = qsq[c * QH:(c + 1) * QH]
        dists, ids = _phase1(qc, qsqc, kT, ksq_p)
        outs.append(_phase2_sc(dists, ids))

    outv = jnp.concatenate([o[0].reshape(QH, 16) for o in outs], axis=0)
    outi = jnp.concatenate([o[1].reshape(QH, 16) for o in outs], axis=0)
    return (outv[:, :10], outi[:, :10])
